# Initial kernel scaffold; baseline (speedup 1.0000x reference)
import jax
import jax.numpy as jnp
from jax.experimental import pallas as pl


def kernel(h, Wr, W1, W2):
    raise NotImplementedError("write your pallas kernel here")


# speedup vs baseline: 2.3283x; 2.3283x over previous
import jax
import jax.numpy as jnp
from jax.experimental import pallas as pl
from jax.experimental.pallas import tpu as pltpu

_B, _S, _H, _E, _F = 1, 2048, 1024, 8, 2048
_TT = 1024
_NT = _S // _TT


def _moe_dense_kernel(h_ref, wr_ref, w1_ref, w2_ref, out_ref, x16_ref, wmat_ref):
    e = pl.program_id(1)

    @pl.when(e == 0)
    def _router():
        hx = h_ref[...]
        x16_ref[...] = hx.astype(jnp.bfloat16)
        logits = jax.lax.dot_general(
            hx, wr_ref[...], (((1,), (1,)), ((), ())),
            preferred_element_type=jnp.float32)
        g = jax.nn.softmax(logits, axis=-1)
        ids = jax.lax.broadcasted_iota(jnp.int32, (_TT, _E), 1)
        gmax = jnp.max(g, axis=1, keepdims=True)
        idx1 = jnp.min(jnp.where(g == gmax, ids, _E), axis=1, keepdims=True)
        oh1 = ids == idx1
        g2 = jnp.where(oh1, -jnp.inf, g)
        gmax2 = jnp.max(g2, axis=1, keepdims=True)
        idx2 = jnp.min(jnp.where(g2 == gmax2, ids, _E), axis=1, keepdims=True)
        oh2 = ids == idx2
        w1v = jnp.sum(jnp.where(oh1, g, 0.0), axis=1, keepdims=True)
        w2v = jnp.sum(jnp.where(oh2, g, 0.0), axis=1, keepdims=True)
        denom = w1v + w2v
        wmat_ref[...] = (jnp.where(oh1, w1v, 0.0)
                         + jnp.where(oh2, w2v, 0.0)) / denom

    x16 = x16_ref[...]
    mid = jax.lax.dot_general(
        x16, w1_ref[0], (((1,), (1,)), ((), ())),
        preferred_element_type=jnp.float32)
    mid = jnp.maximum(mid, 0.0).astype(jnp.bfloat16)
    y = jax.lax.dot_general(
        mid, w2_ref[0], (((1,), (1,)), ((), ())),
        preferred_element_type=jnp.float32)

    onehot = (jax.lax.broadcasted_iota(jnp.int32, (_E, 1), 0) == e)
    w_col = jax.lax.dot_general(
        wmat_ref[...], onehot.astype(jnp.float32), (((1,), (0,)), ((), ())),
        preferred_element_type=jnp.float32)
    contrib = y * w_col

    @pl.when(e == 0)
    def _init():
        out_ref[...] = contrib

    @pl.when(e > 0)
    def _acc():
        out_ref[...] = out_ref[...] + contrib


def kernel(h, Wr, W1, W2):
    h2 = h.reshape(_S, _H)
    w1b = W1.astype(jnp.bfloat16)
    w2b = W2.astype(jnp.bfloat16)
    out = pl.pallas_call(
        _moe_dense_kernel,
        grid=(_NT, _E),
        in_specs=[
            pl.BlockSpec((_TT, _H), lambda t, e: (t, 0)),
            pl.BlockSpec((_E, _H), lambda t, e: (0, 0)),
            pl.BlockSpec((1, _F, _H), lambda t, e: (e, 0, 0)),
            pl.BlockSpec((1, _H, _F), lambda t, e: (e, 0, 0)),
        ],
        out_specs=pl.BlockSpec((_TT, _H), lambda t, e: (t, 0)),
        out_shape=jax.ShapeDtypeStruct((_S, _H), jnp.float32),
        scratch_shapes=[
            pltpu.VMEM((_TT, _H), jnp.bfloat16),
            pltpu.VMEM((_TT, _E), jnp.float32),
        ],
        compiler_params=pltpu.CompilerParams(
            dimension_semantics=("arbitrary", "arbitrary"),
        ),
    )(h2, Wr, w1b, w2b)
    return out.reshape(_B, _S, _H)
